# baseline (speedup 1.0000x reference)
import functools
import numpy as np
import jax
import jax.numpy as jnp
from jax import lax
from jax.experimental import pallas as pl
from jax.experimental.pallas import tpu as pltpu
from jax.experimental.pallas import tpu_sc as plsc

N = 10000
E = 320000
D = 128
NRBF = 16
CUTOFF = 5.0

EB = 512
NB = 2000

_CENTERS = np.linspace(0.5, CUTOFF, NRBF, dtype=np.float32)

NCORE = 2
NSUB = 16
NWORK = NCORE * NSUB
EPW = E // NWORK
ECH = 64
NECH = E // ECH
NLOC = (NECH + NWORK - 1) // NWORK
NPAD = 10240
NROWS = NPAD // NSUB

_sc_mesh = plsc.VectorSubcoreMesh(core_axis_name="c", subcore_axis_name="s")


@functools.partial(
    pl.kernel, mesh=_sc_mesh,
    out_type=jax.ShapeDtypeStruct((NCORE, NPAD, D), jnp.float32),
    scratch_types=[
        pltpu.VMEM((ECH,), jnp.int32),
        pltpu.VMEM((ECH,), jnp.int32),
        pltpu.VMEM((ECH, D), jnp.float32),
        pltpu.VMEM((ECH, D), jnp.float32),
        pltpu.VMEM_SHARED((NPAD, D), jnp.float32),
        pltpu.SemaphoreType.DMA,
    ],
)
def _fwd_edge_sc(h_hbm, fm_hbm, src_hbm, dst_hbm, zeros_hbm, agg_hbm,
                 src_v, dst_v, rows_v, fm_v, agg_sp, gsem):
    c = lax.axis_index("c")
    s = lax.axis_index("s")
    wid = s * NCORE + c

    pltpu.sync_copy(zeros_hbm, agg_sp.at[pl.ds(s * NROWS, NROWS)])
    plsc.subcore_barrier()

    def chunk(i, carry):
        g = i * NWORK + wid

        @pl.when(g < NECH)
        def _():
            off = g * ECH
            pltpu.sync_copy(src_hbm.at[pl.ds(off, ECH)], src_v)
            pltpu.sync_copy(dst_hbm.at[pl.ds(off, ECH)], dst_v)
            pltpu.async_copy(h_hbm.at[src_v], rows_v, gsem).wait()
            pltpu.sync_copy(fm_hbm.at[pl.ds(off, ECH)], fm_v)

            def mult(j, carry2):
                for k in range(D // 16):
                    sl = pl.ds(k * 16, 16)
                    rows_v[j, sl] = rows_v[j, sl] * fm_v[j, sl]
                return carry2
            lax.fori_loop(0, ECH, mult, 0)

            pltpu.sync_copy(rows_v, agg_sp.at[dst_v], add=True)
        return carry
    lax.fori_loop(0, NLOC, chunk, 0)

    plsc.subcore_barrier()
    sl = pl.ds(s * NROWS, NROWS)
    pltpu.sync_copy(agg_sp.at[sl], agg_hbm.at[c, sl])


@functools.partial(
    pl.kernel, mesh=_sc_mesh,
    out_type=jax.ShapeDtypeStruct((E, D), jnp.float32),
    scratch_types=[
        pltpu.VMEM((ECH,), jnp.int32),
        pltpu.VMEM((ECH,), jnp.int32),
        pltpu.VMEM((ECH, D), jnp.float32),
        pltpu.VMEM((ECH, D), jnp.float32),
        pltpu.SemaphoreType.DMA,
        pltpu.SemaphoreType.DMA,
    ],
)
def _bwd_edge_sc(h_hbm, b_hbm, src_hbm, dst_hbm, s_hbm,
                 src_v, dst_v, rows_v, brows_v, gsem, gsem2):
    c = lax.axis_index("c")
    s = lax.axis_index("s")
    wid = s * NCORE + c

    def chunk(i, carry):
        g = i * NWORK + wid

        @pl.when(g < NECH)
        def _():
            off = g * ECH
            pltpu.sync_copy(src_hbm.at[pl.ds(off, ECH)], src_v)
            pltpu.sync_copy(dst_hbm.at[pl.ds(off, ECH)], dst_v)
            cp1 = pltpu.async_copy(h_hbm.at[src_v], rows_v, gsem)
            cp2 = pltpu.async_copy(b_hbm.at[dst_v], brows_v, gsem2)
            cp1.wait()
            cp2.wait()

            def mult(j, carry2):
                for k in range(D // 16):
                    sl = pl.ds(k * 16, 16)
                    rows_v[j, sl] = rows_v[j, sl] * brows_v[j, sl]
                return carry2
            lax.fori_loop(0, ECH, mult, 0)

            pltpu.sync_copy(rows_v, s_hbm.at[pl.ds(off, ECH)])
        return carry
    lax.fori_loop(0, NLOC, chunk, 0)


@functools.partial(
    pl.kernel, mesh=_sc_mesh,
    out_type=[jax.ShapeDtypeStruct((E,), jnp.float32) for _ in range(5)],
    compiler_params=pltpu.CompilerParams(needs_layout_passes=False),
    scratch_types=(
        [pltpu.VMEM((ECH,), jnp.int32) for _ in range(2)]
        + [pltpu.VMEM((ECH,), jnp.float32) for _ in range(11)]
        + [pltpu.SemaphoreType.DMA]
    ),
)
def _geom_sc(px_hbm, py_hbm, pz_hbm, src_hbm, dst_hbm,
             r_hbm, invr_hbm, vx_hbm, vy_hbm, vz_hbm,
             src_v, dst_v, pxs_v, pxd_v, pys_v, pyd_v, pzs_v, pzd_v,
             r_v, ivr_v, vx_v, vy_v, vz_v, gsem):
    c = lax.axis_index("c")
    s = lax.axis_index("s")
    wid = s * NCORE + c

    def chunk(i, carry):
        g = i * NWORK + wid

        @pl.when(g < NECH)
        def _():
            off = g * ECH
            pltpu.sync_copy(src_hbm.at[pl.ds(off, ECH)], src_v)
            pltpu.sync_copy(dst_hbm.at[pl.ds(off, ECH)], dst_v)
            for tab, idx, dstv in ((px_hbm, src_v, pxs_v), (px_hbm, dst_v, pxd_v),
                                   (py_hbm, src_v, pys_v), (py_hbm, dst_v, pyd_v),
                                   (pz_hbm, src_v, pzs_v), (pz_hbm, dst_v, pzd_v)):
                pltpu.async_copy(tab.at[idx], dstv, gsem).wait()
            for k in range(ECH // 16):
                sl = pl.ds(k * 16, 16)
                vx = pxd_v[sl] - pxs_v[sl]
                vy = pyd_v[sl] - pys_v[sl]
                vz = pzd_v[sl] - pzs_v[sl]
                r2 = vx * vx + vy * vy + vz * vz + 1e-12
                y = plsc.bitcast(
                    jnp.int32(0x5F3759DF) - (plsc.bitcast(r2, jnp.int32) >> 1),
                    jnp.float32)
                half = 0.5 * r2
                for _ in range(3):
                    y = y * (1.5 - half * y * y)
                r_v[sl] = r2 * y
                ivr_v[sl] = y
                vx_v[sl] = vx
                vy_v[sl] = vy
                vz_v[sl] = vz
            pltpu.sync_copy(r_v, r_hbm.at[pl.ds(off, ECH)])
            pltpu.sync_copy(ivr_v, invr_hbm.at[pl.ds(off, ECH)])
            pltpu.sync_copy(vx_v, vx_hbm.at[pl.ds(off, ECH)])
            pltpu.sync_copy(vy_v, vy_hbm.at[pl.ds(off, ECH)])
            pltpu.sync_copy(vz_v, vz_hbm.at[pl.ds(off, ECH)])
        return carry
    lax.fori_loop(0, NLOC, chunk, 0)


FCH = 64
NFCH = E // FCH


@functools.partial(
    pl.kernel, mesh=_sc_mesh,
    out_type=jax.ShapeDtypeStruct((NCORE, NPAD // 8, 128), jnp.float32),
    scratch_types=[
        pltpu.VMEM((FCH,), jnp.int32),
        pltpu.VMEM((FCH,), jnp.int32),
        pltpu.VMEM((FCH // 8, 128), jnp.float32),
        pltpu.VMEM((FCH, 16), jnp.float32),
        pltpu.VMEM((FCH, 16), jnp.float32),
        pltpu.VMEM_SHARED((NPAD, 16), jnp.float32),
    ],
)
def _force_scatter_sc(w8_hbm, src_hbm, dst_hbm, f_hbm,
                      src_v, dst_v, w8_v, w_v, wneg_v, f_sp):
    c = lax.axis_index("c")
    s = lax.axis_index("s")
    wid = s * NCORE + c

    def z(j, carry):
        w_v[j, :] = jnp.zeros((16,), jnp.float32)
        return carry
    lax.fori_loop(0, FCH, z, 0)

    def zcp(j, carry):
        pltpu.sync_copy(w_v, f_sp.at[pl.ds(s * NROWS + j * FCH, FCH)])
        return carry
    lax.fori_loop(0, NROWS // FCH, zcp, 0)
    plsc.subcore_barrier()

    nloc = (NFCH + NWORK - 1) // NWORK

    def chunk(i, carry):
        g = i * NWORK + wid

        @pl.when(g < NFCH)
        def _():
            off = g * FCH
            pltpu.sync_copy(src_hbm.at[pl.ds(off, FCH)], src_v)
            pltpu.sync_copy(dst_hbm.at[pl.ds(off, FCH)], dst_v)
            pltpu.sync_copy(w8_hbm.at[pl.ds(g * (FCH // 8), FCH // 8)], w8_v)

            def rp(j, c2):
                for k in range(8):
                    vv = w8_v[j, pl.ds(k * 16, 16)]
                    w_v[j * 8 + k, :] = vv
                    wneg_v[j * 8 + k, :] = -vv
                return c2
            lax.fori_loop(0, FCH // 8, rp, 0)

            pltpu.sync_copy(w_v, f_sp.at[src_v], add=True)
            pltpu.sync_copy(wneg_v, f_sp.at[dst_v], add=True)
        return carry
    lax.fori_loop(0, nloc, chunk, 0)

    plsc.subcore_barrier()

    def dump(j, carry):
        pltpu.sync_copy(f_sp.at[pl.ds(s * NROWS + j * FCH, FCH)], w_v)

        def pk(m, c2):
            for k in range(8):
                w8_v[m, pl.ds(k * 16, 16)] = w_v[m * 8 + k, :]
            return c2
        lax.fori_loop(0, FCH // 8, pk, 0)
        pltpu.sync_copy(
            w8_v, f_hbm.at[c, pl.ds(s * (NROWS // 8) + j * (FCH // 8), FCH // 8)])
        return carry
    lax.fori_loop(0, NROWS // FCH, dump, 0)


def _embed_body(an_ref, embed_ref, h_ref):
    an = an_ref[...]
    z = jax.lax.broadcasted_iota(jnp.int32, (1, 128), 1)
    onehot = (an == z).astype(jnp.float32)
    h_ref[...] = jnp.dot(onehot, embed_ref[...],
                         preferred_element_type=jnp.float32)


def _rbf_body(r_ref, wrbf_ref, fm_ref, rp_ref):
    r = r_ref[...]
    step = (CUTOFF - 0.5) / (NRBF - 1)
    centers = 0.5 + step * jax.lax.broadcasted_iota(
        jnp.int32, (1, NRBF), 1).astype(jnp.float32)
    dkr = r - centers
    g = jnp.exp(-2.0 * dkr * dkr)
    x = r / CUTOFF
    inside = x < 1.0
    env = jnp.where(inside, 0.5 * (jnp.cos(jnp.pi * x) + 1.0), 0.0)
    envp = jnp.where(inside, -(jnp.pi / (2.0 * CUTOFF)) * jnp.sin(jnp.pi * x), 0.0)
    R = g * env
    rp_ref[...] = (-4.0 * dkr * g) * env + g * envp
    fm_ref[...] = jnp.dot(R, wrbf_ref[...], preferred_element_type=jnp.float32)


def _node_body(h_ref, agg_ref, w0_ref, w1_ref, wout_ref, b_ref):
    h = h_ref[...]
    agg = agg_ref[...]
    u = (jnp.dot(h, w0_ref[...], preferred_element_type=jnp.float32)
         + jnp.dot(agg, w1_ref[...], preferred_element_type=jnp.float32))
    sig = jax.nn.sigmoid(u)
    silup = sig * (1.0 + u * (1.0 - sig))
    a = silup * wout_ref[...]
    b_ref[...] = jax.lax.dot_general(
        a, w1_ref[...], (((1,), (1,)), ((), ())),
        preferred_element_type=jnp.float32)


def _edge_back_body(s_ref, rp_ref, v16_ref, invr_ref, wrbf_ref, w_ref):
    s = s_ref[...]
    q = jax.lax.dot_general(
        s, wrbf_ref[...], (((1,), (1,)), ((), ())),
        preferred_element_type=jnp.float32)
    t = jnp.sum(q * rp_ref[...], axis=1, keepdims=True)
    w_ref[...] = (t * invr_ref[...]) * v16_ref[...]


def kernel(positions, cell, shifts_idx, edge_index, atomic_numbers, embed, W_rbf, W0, W1, w_out):
    src = edge_index[0].astype(jnp.int32)
    dst = edge_index[1].astype(jnp.int32)
    an = atomic_numbers.astype(jnp.int32)

    embed_pad = jnp.zeros((128, 128), jnp.float32).at[:embed.shape[0]].set(embed)
    h = pl.pallas_call(
        _embed_body,
        grid=(N // NB,),
        in_specs=[pl.BlockSpec((NB, 1), lambda i: (i, 0)),
                  pl.BlockSpec((128, 128), lambda i: (0, 0))],
        out_specs=pl.BlockSpec((NB, 128), lambda i: (i, 0)),
        out_shape=jax.ShapeDtypeStruct((N, 128), jnp.float32),
    )(an[:, None], embed_pad)

    px = jnp.asarray(positions[:, 0])
    py = jnp.asarray(positions[:, 1])
    pz = jnp.asarray(positions[:, 2])
    r, invr, vx, vy, vz = _geom_sc(px, py, pz, src, dst)

    Fm, Rp = pl.pallas_call(
        _rbf_body,
        grid=(E // EB,),
        in_specs=[pl.BlockSpec((EB, 1), lambda i: (i, 0)),
                  pl.BlockSpec((NRBF, 128), lambda i: (0, 0))],
        out_specs=[pl.BlockSpec((EB, 128), lambda i: (i, 0)),
                   pl.BlockSpec((EB, NRBF), lambda i: (i, 0))],
        out_shape=[jax.ShapeDtypeStruct((E, 128), jnp.float32),
                   jax.ShapeDtypeStruct((E, NRBF), jnp.float32)],
    )(r[:, None], W_rbf)

    zeros_rows = jnp.zeros((NROWS, D), jnp.float32)
    agg2 = _fwd_edge_sc(h, Fm, src, dst, zeros_rows)
    agg = agg2[0, :N] + agg2[1, :N]

    b = pl.pallas_call(
        _node_body,
        grid=(N // NB,),
        in_specs=[pl.BlockSpec((NB, 128), lambda i: (i, 0)),
                  pl.BlockSpec((NB, 128), lambda i: (i, 0)),
                  pl.BlockSpec((128, 128), lambda i: (0, 0)),
                  pl.BlockSpec((128, 128), lambda i: (0, 0)),
                  pl.BlockSpec((1, 128), lambda i: (0, 0))],
        out_specs=pl.BlockSpec((NB, 128), lambda i: (i, 0)),
        out_shape=jax.ShapeDtypeStruct((N, 128), jnp.float32),
    )(h, agg, W0, W1, w_out[None, :])

    s_arr = _bwd_edge_sc(h, b, src, dst)

    v16 = jnp.zeros((E, 16), jnp.float32)
    v16 = v16.at[:, 0].set(vx).at[:, 1].set(vy).at[:, 2].set(vz)
    w_rows = pl.pallas_call(
        _edge_back_body,
        grid=(E // EB,),
        in_specs=[pl.BlockSpec((EB, 128), lambda i: (i, 0)),
                  pl.BlockSpec((EB, NRBF), lambda i: (i, 0)),
                  pl.BlockSpec((EB, 16), lambda i: (i, 0)),
                  pl.BlockSpec((EB, 1), lambda i: (i, 0)),
                  pl.BlockSpec((NRBF, 128), lambda i: (0, 0))],
        out_specs=pl.BlockSpec((EB, 16), lambda i: (i, 0)),
        out_shape=jax.ShapeDtypeStruct((E, 16), jnp.float32),
    )(s_arr, Rp, v16, invr[:, None], W_rbf)

    w3 = w_rows[:, :3]
    forces = (jax.ops.segment_sum(w3, src, num_segments=N)
              - jax.ops.segment_sum(w3, dst, num_segments=N))
    return forces


# speedup vs baseline: 1.0090x; 1.0090x over previous
import functools
import numpy as np
import jax
import jax.numpy as jnp
from jax import lax
from jax.experimental import pallas as pl
from jax.experimental.pallas import tpu as pltpu
from jax.experimental.pallas import tpu_sc as plsc

N = 10000
E = 320000
D = 128
NRBF = 16
CUTOFF = 5.0

EB = 512
NB = 2000

_CENTERS = np.linspace(0.5, CUTOFF, NRBF, dtype=np.float32)

NCORE = 2
NSUB = 16
NWORK = NCORE * NSUB
EPW = E // NWORK
ECH = 64
NECH = E // ECH
NLOC = (NECH + NWORK - 1) // NWORK
NPAD = 10240
NROWS = NPAD // NSUB

_sc_mesh = plsc.VectorSubcoreMesh(core_axis_name="c", subcore_axis_name="s")


@functools.partial(
    pl.kernel, mesh=_sc_mesh,
    out_type=jax.ShapeDtypeStruct((NCORE, NPAD, D), jnp.float32),
    scratch_types=[
        pltpu.VMEM((2, ECH), jnp.int32),
        pltpu.VMEM((2, ECH), jnp.int32),
        pltpu.VMEM((ECH,), jnp.int32),
        pltpu.VMEM((2, ECH, D), jnp.float32),
        pltpu.VMEM((2, ECH, D), jnp.float32),
        pltpu.VMEM_SHARED((NPAD, D), jnp.float32),
        pltpu.SemaphoreType.DMA((2,)),
        pltpu.SemaphoreType.DMA((2,)),
        pltpu.SemaphoreType.DMA((2,)),
    ],
)
def _fwd_edge_sc(h_hbm, fm_hbm, src_hbm, dst_hbm, zeros_hbm, agg_hbm,
                 src2_v, dst2_v, dstf_v, rows2_v, fm2_v, agg_sp,
                 isem, gsem, fsem):
    c = lax.axis_index("c")
    s = lax.axis_index("s")
    wid = s * NCORE + c

    pltpu.sync_copy(zeros_hbm, agg_sp.at[pl.ds(s * NROWS, NROWS)])
    plsc.subcore_barrier()

    def issue_idx(g, slot):
        off = g * ECH
        pltpu.async_copy(src_hbm.at[pl.ds(off, ECH)], src2_v.at[slot], isem.at[slot])
        pltpu.async_copy(dst_hbm.at[pl.ds(off, ECH)], dst2_v.at[slot], isem.at[slot])

    def drain_idx(slot):
        pltpu.make_async_copy(src_hbm.at[pl.ds(0, ECH)], src2_v.at[slot], isem.at[slot]).wait()
        pltpu.make_async_copy(dst_hbm.at[pl.ds(0, ECH)], dst2_v.at[slot], isem.at[slot]).wait()

    def issue_main(g, slot):
        pltpu.async_copy(h_hbm.at[src2_v.at[slot]], rows2_v.at[slot], gsem.at[slot])
        pltpu.async_copy(fm_hbm.at[pl.ds(g * ECH, ECH)], fm2_v.at[slot], fsem.at[slot])

    def drain_main(slot):
        pltpu.make_async_copy(h_hbm.at[pl.ds(0, ECH)], rows2_v.at[slot], gsem.at[slot]).wait()
        pltpu.make_async_copy(fm_hbm.at[pl.ds(0, ECH)], fm2_v.at[slot], fsem.at[slot]).wait()

    issue_idx(wid, 0)
    issue_idx(NWORK + wid, 1)
    drain_idx(0)
    issue_main(wid, 0)

    def chunk(i, carry):
        @pl.when(i * NWORK + wid < NECH)
        def _():
            par = lax.rem(i, 2)
            nxt = 1 - par
            g1 = (i + 1) * NWORK + wid

            @pl.when(g1 < NECH)
            def _():
                drain_idx(nxt)
                issue_main(g1, nxt)

            drain_main(par)

            g2 = (i + 2) * NWORK + wid

            @pl.when(g2 < NECH)
            def _():
                issue_idx(g2, par)

            for k in range(ECH // 16):
                sl = pl.ds(k * 16, 16)
                dstf_v[sl] = dst2_v[par, sl]

            def mult(j, carry2):
                for k in range(D // 16):
                    sl = pl.ds(k * 16, 16)
                    rows2_v[par, j, sl] = rows2_v[par, j, sl] * fm2_v[par, j, sl]
                return carry2
            lax.fori_loop(0, ECH, mult, 0)

            pltpu.sync_copy(rows2_v.at[par], agg_sp.at[dstf_v], add=True)
        return carry
    lax.fori_loop(0, NLOC, chunk, 0)

    plsc.subcore_barrier()
    sl = pl.ds(s * NROWS, NROWS)
    pltpu.sync_copy(agg_sp.at[sl], agg_hbm.at[c, sl])


@functools.partial(
    pl.kernel, mesh=_sc_mesh,
    out_type=jax.ShapeDtypeStruct((E, D), jnp.float32),
    scratch_types=[
        pltpu.VMEM((ECH,), jnp.int32),
        pltpu.VMEM((ECH,), jnp.int32),
        pltpu.VMEM((ECH, D), jnp.float32),
        pltpu.VMEM((ECH, D), jnp.float32),
        pltpu.SemaphoreType.DMA,
        pltpu.SemaphoreType.DMA,
    ],
)
def _bwd_edge_sc(h_hbm, b_hbm, src_hbm, dst_hbm, s_hbm,
                 src_v, dst_v, rows_v, brows_v, gsem, gsem2):
    c = lax.axis_index("c")
    s = lax.axis_index("s")
    wid = s * NCORE + c

    def chunk(i, carry):
        g = i * NWORK + wid

        @pl.when(g < NECH)
        def _():
            off = g * ECH
            pltpu.sync_copy(src_hbm.at[pl.ds(off, ECH)], src_v)
            pltpu.sync_copy(dst_hbm.at[pl.ds(off, ECH)], dst_v)
            cp1 = pltpu.async_copy(h_hbm.at[src_v], rows_v, gsem)
            cp2 = pltpu.async_copy(b_hbm.at[dst_v], brows_v, gsem2)
            cp1.wait()
            cp2.wait()

            def mult(j, carry2):
                for k in range(D // 16):
                    sl = pl.ds(k * 16, 16)
                    rows_v[j, sl] = rows_v[j, sl] * brows_v[j, sl]
                return carry2
            lax.fori_loop(0, ECH, mult, 0)

            pltpu.sync_copy(rows_v, s_hbm.at[pl.ds(off, ECH)])
        return carry
    lax.fori_loop(0, NLOC, chunk, 0)


@functools.partial(
    pl.kernel, mesh=_sc_mesh,
    out_type=[jax.ShapeDtypeStruct((E,), jnp.float32) for _ in range(5)],
    compiler_params=pltpu.CompilerParams(needs_layout_passes=False),
    scratch_types=(
        [pltpu.VMEM((ECH,), jnp.int32) for _ in range(2)]
        + [pltpu.VMEM((ECH,), jnp.float32) for _ in range(11)]
        + [pltpu.SemaphoreType.DMA]
    ),
)
def _geom_sc(px_hbm, py_hbm, pz_hbm, src_hbm, dst_hbm,
             r_hbm, invr_hbm, vx_hbm, vy_hbm, vz_hbm,
             src_v, dst_v, pxs_v, pxd_v, pys_v, pyd_v, pzs_v, pzd_v,
             r_v, ivr_v, vx_v, vy_v, vz_v, gsem):
    c = lax.axis_index("c")
    s = lax.axis_index("s")
    wid = s * NCORE + c

    def chunk(i, carry):
        g = i * NWORK + wid

        @pl.when(g < NECH)
        def _():
            off = g * ECH
            pltpu.sync_copy(src_hbm.at[pl.ds(off, ECH)], src_v)
            pltpu.sync_copy(dst_hbm.at[pl.ds(off, ECH)], dst_v)
            for tab, idx, dstv in ((px_hbm, src_v, pxs_v), (px_hbm, dst_v, pxd_v),
                                   (py_hbm, src_v, pys_v), (py_hbm, dst_v, pyd_v),
                                   (pz_hbm, src_v, pzs_v), (pz_hbm, dst_v, pzd_v)):
                pltpu.async_copy(tab.at[idx], dstv, gsem).wait()
            for k in range(ECH // 16):
                sl = pl.ds(k * 16, 16)
                vx = pxd_v[sl] - pxs_v[sl]
                vy = pyd_v[sl] - pys_v[sl]
                vz = pzd_v[sl] - pzs_v[sl]
                r2 = vx * vx + vy * vy + vz * vz + 1e-12
                y = plsc.bitcast(
                    jnp.int32(0x5F3759DF) - (plsc.bitcast(r2, jnp.int32) >> 1),
                    jnp.float32)
                half = 0.5 * r2
                for _ in range(3):
                    y = y * (1.5 - half * y * y)
                r_v[sl] = r2 * y
                ivr_v[sl] = y
                vx_v[sl] = vx
                vy_v[sl] = vy
                vz_v[sl] = vz
            pltpu.sync_copy(r_v, r_hbm.at[pl.ds(off, ECH)])
            pltpu.sync_copy(ivr_v, invr_hbm.at[pl.ds(off, ECH)])
            pltpu.sync_copy(vx_v, vx_hbm.at[pl.ds(off, ECH)])
            pltpu.sync_copy(vy_v, vy_hbm.at[pl.ds(off, ECH)])
            pltpu.sync_copy(vz_v, vz_hbm.at[pl.ds(off, ECH)])
        return carry
    lax.fori_loop(0, NLOC, chunk, 0)


FCH = 64
NFCH = E // FCH


@functools.partial(
    pl.kernel, mesh=_sc_mesh,
    out_type=jax.ShapeDtypeStruct((NCORE, NPAD // 8, 128), jnp.float32),
    scratch_types=[
        pltpu.VMEM((FCH,), jnp.int32),
        pltpu.VMEM((FCH,), jnp.int32),
        pltpu.VMEM((FCH // 8, 128), jnp.float32),
        pltpu.VMEM((FCH, 16), jnp.float32),
        pltpu.VMEM((FCH, 16), jnp.float32),
        pltpu.VMEM_SHARED((NPAD, 16), jnp.float32),
    ],
)
def _force_scatter_sc(w8_hbm, src_hbm, dst_hbm, f_hbm,
                      src_v, dst_v, w8_v, w_v, wneg_v, f_sp):
    c = lax.axis_index("c")
    s = lax.axis_index("s")
    wid = s * NCORE + c

    def z(j, carry):
        w_v[j, :] = jnp.zeros((16,), jnp.float32)
        return carry
    lax.fori_loop(0, FCH, z, 0)

    def zcp(j, carry):
        pltpu.sync_copy(w_v, f_sp.at[pl.ds(s * NROWS + j * FCH, FCH)])
        return carry
    lax.fori_loop(0, NROWS // FCH, zcp, 0)
    plsc.subcore_barrier()

    nloc = (NFCH + NWORK - 1) // NWORK

    def chunk(i, carry):
        g = i * NWORK + wid

        @pl.when(g < NFCH)
        def _():
            off = g * FCH
            pltpu.sync_copy(src_hbm.at[pl.ds(off, FCH)], src_v)
            pltpu.sync_copy(dst_hbm.at[pl.ds(off, FCH)], dst_v)
            pltpu.sync_copy(w8_hbm.at[pl.ds(g * (FCH // 8), FCH // 8)], w8_v)

            def rp(j, c2):
                for k in range(8):
                    vv = w8_v[j, pl.ds(k * 16, 16)]
                    w_v[j * 8 + k, :] = vv
                    wneg_v[j * 8 + k, :] = -vv
                return c2
            lax.fori_loop(0, FCH // 8, rp, 0)

            pltpu.sync_copy(w_v, f_sp.at[src_v], add=True)
            pltpu.sync_copy(wneg_v, f_sp.at[dst_v], add=True)
        return carry
    lax.fori_loop(0, nloc, chunk, 0)

    plsc.subcore_barrier()

    def dump(j, carry):
        pltpu.sync_copy(f_sp.at[pl.ds(s * NROWS + j * FCH, FCH)], w_v)

        def pk(m, c2):
            for k in range(8):
                w8_v[m, pl.ds(k * 16, 16)] = w_v[m * 8 + k, :]
            return c2
        lax.fori_loop(0, FCH // 8, pk, 0)
        pltpu.sync_copy(
            w8_v, f_hbm.at[c, pl.ds(s * (NROWS // 8) + j * (FCH // 8), FCH // 8)])
        return carry
    lax.fori_loop(0, NROWS // FCH, dump, 0)


def _embed_body(an_ref, embed_ref, h_ref):
    an = an_ref[...]
    z = jax.lax.broadcasted_iota(jnp.int32, (1, 128), 1)
    onehot = (an == z).astype(jnp.float32)
    h_ref[...] = jnp.dot(onehot, embed_ref[...],
                         preferred_element_type=jnp.float32)


def _rbf_body(r_ref, wrbf_ref, fm_ref, rp_ref):
    r = r_ref[...]
    step = (CUTOFF - 0.5) / (NRBF - 1)
    centers = 0.5 + step * jax.lax.broadcasted_iota(
        jnp.int32, (1, NRBF), 1).astype(jnp.float32)
    dkr = r - centers
    g = jnp.exp(-2.0 * dkr * dkr)
    x = r / CUTOFF
    inside = x < 1.0
    env = jnp.where(inside, 0.5 * (jnp.cos(jnp.pi * x) + 1.0), 0.0)
    envp = jnp.where(inside, -(jnp.pi / (2.0 * CUTOFF)) * jnp.sin(jnp.pi * x), 0.0)
    R = g * env
    rp_ref[...] = (-4.0 * dkr * g) * env + g * envp
    fm_ref[...] = jnp.dot(R, wrbf_ref[...], preferred_element_type=jnp.float32)


def _node_body(h_ref, agg_ref, w0_ref, w1_ref, wout_ref, b_ref):
    h = h_ref[...]
    agg = agg_ref[...]
    u = (jnp.dot(h, w0_ref[...], preferred_element_type=jnp.float32)
         + jnp.dot(agg, w1_ref[...], preferred_element_type=jnp.float32))
    sig = jax.nn.sigmoid(u)
    silup = sig * (1.0 + u * (1.0 - sig))
    a = silup * wout_ref[...]
    b_ref[...] = jax.lax.dot_general(
        a, w1_ref[...], (((1,), (1,)), ((), ())),
        preferred_element_type=jnp.float32)


def _edge_back_body(s_ref, rp_ref, v16_ref, invr_ref, wrbf_ref, w_ref):
    s = s_ref[...]
    q = jax.lax.dot_general(
        s, wrbf_ref[...], (((1,), (1,)), ((), ())),
        preferred_element_type=jnp.float32)
    t = jnp.sum(q * rp_ref[...], axis=1, keepdims=True)
    w_ref[...] = (t * invr_ref[...]) * v16_ref[...]


def kernel(positions, cell, shifts_idx, edge_index, atomic_numbers, embed, W_rbf, W0, W1, w_out):
    src = edge_index[0].astype(jnp.int32)
    dst = edge_index[1].astype(jnp.int32)
    an = atomic_numbers.astype(jnp.int32)

    embed_pad = jnp.zeros((128, 128), jnp.float32).at[:embed.shape[0]].set(embed)
    h = pl.pallas_call(
        _embed_body,
        grid=(N // NB,),
        in_specs=[pl.BlockSpec((NB, 1), lambda i: (i, 0)),
                  pl.BlockSpec((128, 128), lambda i: (0, 0))],
        out_specs=pl.BlockSpec((NB, 128), lambda i: (i, 0)),
        out_shape=jax.ShapeDtypeStruct((N, 128), jnp.float32),
    )(an[:, None], embed_pad)

    px = jnp.asarray(positions[:, 0])
    py = jnp.asarray(positions[:, 1])
    pz = jnp.asarray(positions[:, 2])
    r, invr, vx, vy, vz = _geom_sc(px, py, pz, src, dst)

    Fm, Rp = pl.pallas_call(
        _rbf_body,
        grid=(E // EB,),
        in_specs=[pl.BlockSpec((EB, 1), lambda i: (i, 0)),
                  pl.BlockSpec((NRBF, 128), lambda i: (0, 0))],
        out_specs=[pl.BlockSpec((EB, 128), lambda i: (i, 0)),
                   pl.BlockSpec((EB, NRBF), lambda i: (i, 0))],
        out_shape=[jax.ShapeDtypeStruct((E, 128), jnp.float32),
                   jax.ShapeDtypeStruct((E, NRBF), jnp.float32)],
    )(r[:, None], W_rbf)

    zeros_rows = jnp.zeros((NROWS, D), jnp.float32)
    agg2 = _fwd_edge_sc(h, Fm, src, dst, zeros_rows)
    agg = agg2[0, :N] + agg2[1, :N]

    b = pl.pallas_call(
        _node_body,
        grid=(N // NB,),
        in_specs=[pl.BlockSpec((NB, 128), lambda i: (i, 0)),
                  pl.BlockSpec((NB, 128), lambda i: (i, 0)),
                  pl.BlockSpec((128, 128), lambda i: (0, 0)),
                  pl.BlockSpec((128, 128), lambda i: (0, 0)),
                  pl.BlockSpec((1, 128), lambda i: (0, 0))],
        out_specs=pl.BlockSpec((NB, 128), lambda i: (i, 0)),
        out_shape=jax.ShapeDtypeStruct((N, 128), jnp.float32),
    )(h, agg, W0, W1, w_out[None, :])

    s_arr = _bwd_edge_sc(h, b, src, dst)

    v16 = jnp.zeros((E, 16), jnp.float32)
    v16 = v16.at[:, 0].set(vx).at[:, 1].set(vy).at[:, 2].set(vz)
    w_rows = pl.pallas_call(
        _edge_back_body,
        grid=(E // EB,),
        in_specs=[pl.BlockSpec((EB, 128), lambda i: (i, 0)),
                  pl.BlockSpec((EB, NRBF), lambda i: (i, 0)),
                  pl.BlockSpec((EB, 16), lambda i: (i, 0)),
                  pl.BlockSpec((EB, 1), lambda i: (i, 0)),
                  pl.BlockSpec((NRBF, 128), lambda i: (0, 0))],
        out_specs=pl.BlockSpec((EB, 16), lambda i: (i, 0)),
        out_shape=jax.ShapeDtypeStruct((E, 16), jnp.float32),
    )(s_arr, Rp, v16, invr[:, None], W_rbf)

    w3 = w_rows[:, :3]
    forces = (jax.ops.segment_sum(w3, src, num_segments=N)
              - jax.ops.segment_sum(w3, dst, num_segments=N))
    return forces
